# baseline (speedup 1.0000x reference)
import functools

import jax
import jax.numpy as jnp
from jax import lax
from jax.experimental import pallas as pl
from jax.experimental.pallas import tpu as pltpu
from jax.experimental.pallas import tpu_sc as plsc

N = 10000
D = 128
B = 128
NC = 2
NS = 16
NW = NC * NS
L = 16
CB = D // L
GROUPS = N // L
GBASE = GROUPS // NW
GEXTRA = GROUPS - GBASE * NW
RBASE = GBASE * L
RMAX = (GBASE + 1) * L
CNTB = B
DUMP = 2 * B
SROWS = 264
CHUNK = 4


def _iota():
    return lax.iota(jnp.int32, L)


_mesh = plsc.VectorSubcoreMesh(core_axis_name="c", subcore_axis_name="s")


@functools.partial(
    pl.kernel,
    compiler_params=pltpu.CompilerParams(needs_layout_passes=False),
    out_type=jax.ShapeDtypeStruct((NC, 2 * B, D), jnp.float32),
    mesh=_mesh,
    scratch_types=[
        pltpu.VMEM((RMAX, D), jnp.float32),
        pltpu.VMEM((RMAX + L,), jnp.int32),
        pltpu.VMEM((L, D), jnp.float32),
        pltpu.VMEM((L,), jnp.int32),
        pltpu.VMEM((8, D), jnp.float32),
        pltpu.SMEM((4,), jnp.int32),
        pltpu.SemaphoreType.DMA,
        pltpu.SemaphoreType.DMA,
        pltpu.VMEM_SHARED((SROWS, D), jnp.float32),
    ],
)
def _seg_sums(x_hbm, batch_hbm, out_hbm,
              xbuf, bbuf, flushbuf, idbuf, zbuf, sm, semx, semb, sacc):
    cid = lax.axis_index("c")
    sid = lax.axis_index("s")
    w = cid * NS + sid

    zero = jnp.zeros((L,), jnp.float32)
    dump_idv = jnp.full((L,), DUMP, jnp.int32)

    base = (GBASE * w + jnp.minimum(w, GEXTRA)) * L
    nfull = RBASE // (CHUNK * L)
    for k in range(nfull):
        pltpu.async_copy(x_hbm.at[pl.ds(base + k * CHUNK * L, CHUNK * L)],
                         xbuf.at[pl.ds(k * CHUNK * L, CHUNK * L)], semx)
    TAIL = RBASE - nfull * CHUNK * L
    pltpu.async_copy(x_hbm.at[pl.ds(base + nfull * CHUNK * L, TAIL)],
                     xbuf.at[pl.ds(nfull * CHUNK * L, TAIL)], semx)

    @pl.when(w < GEXTRA)
    def _():
        pltpu.async_copy(x_hbm.at[pl.ds(base + RBASE, L)],
                         xbuf.at[pl.ds(RBASE, L)], semx)
        pltpu.async_copy(batch_hbm.at[pl.ds(base + RBASE, L)],
                         bbuf.at[pl.ds(RBASE, L)], semb)

    bcp = pltpu.async_copy(batch_hbm.at[pl.ds(base, RBASE)],
                           bbuf.at[pl.ds(0, RBASE)], semb)

    for r in range(8):
        for cc in range(CB):
            zbuf[r, pl.ds(cc * L, L)] = zero
    sm[0] = jnp.int32(-1)
    sm[1] = jnp.int32(0)
    idbuf[...] = dump_idv

    pltpu.sync_copy(zbuf, sacc.at[pl.ds(sid * 16, 8)])
    pltpu.sync_copy(zbuf, sacc.at[pl.ds(sid * 16 + 8, 8)])

    @pl.when(sid == 0)
    def _():
        pltpu.sync_copy(zbuf, sacc.at[pl.ds(256, 8)])

    plsc.subcore_barrier()
    bcp.wait()

    def do_flush(seg, accs):
        row_s = sm[1] % L
        rv0 = jnp.full((L,), row_s, jnp.int32)
        rv1 = rv0 + 1
        for cc in range(CB):
            plsc.store_scatter(flushbuf, [rv0, cc * L + _iota()], accs[cc])
            plsc.store_scatter(flushbuf, [rv1, cc * L + _iota()], accs[CB])
        idv = idbuf[...]
        idv = jnp.where(_iota() == row_s, jnp.full((L,), seg, jnp.int32), idv)
        idv = jnp.where(_iota() == row_s + 1,
                        jnp.full((L,), seg + CNTB, jnp.int32), idv)
        idbuf[...] = idv

        @pl.when(row_s == L - 2)
        def _():
            pltpu.sync_copy(flushbuf, sacc.at[idbuf], add=True)
            idbuf[...] = dump_idv

        sm[1] = sm[1] + 2

    def step(g, accs):
        accs = list(accs)
        rbase = g * L

        @pl.when((g < nfull * CHUNK) & (g % CHUNK == 0))
        def _():
            pltpu.make_async_copy(x_hbm.at[pl.ds(0, CHUNK * L)],
                                  xbuf.at[pl.ds(0, CHUNK * L)], semx).wait()

        @pl.when(g == nfull * CHUNK)
        def _():
            pltpu.make_async_copy(x_hbm.at[pl.ds(0, TAIL)],
                                  xbuf.at[pl.ds(0, TAIL)], semx).wait()

        @pl.when(g == GBASE)
        def _():
            pltpu.make_async_copy(x_hbm.at[pl.ds(0, L)],
                                  xbuf.at[pl.ds(0, L)], semx).wait()
            pltpu.make_async_copy(batch_hbm.at[pl.ds(0, L)],
                                  bbuf.at[pl.ds(0, L)], semb).wait()

        bv = bbuf[pl.ds(rbase, L)]
        b0 = bv[0]
        b15 = bv[L - 1]
        cur0 = sm[0]

        def fastf(accs):
            @pl.when(cur0 != b0)
            def _():
                @pl.when(cur0 >= 0)
                def _():
                    do_flush(cur0, accs)

            bvp = jnp.full((L,), cur0 != b0)
            run = [jnp.where(bvp, 0.0, a) for a in accs]
            for r in range(L):
                for cc in range(CB):
                    run[cc] = run[cc] + xbuf[rbase + r, pl.ds(cc * L, L)]
            run[CB] = run[CB] + jnp.float32(L)
            return tuple(run)

        def slowf(accs):
            def row(r, accs):
                accs = list(accs)
                br = bbuf[pl.ds(rbase + r, L)][0]
                prev = sm[0]
                changed = prev != br

                @pl.when(changed)
                def _():
                    @pl.when(prev >= 0)
                    def _():
                        do_flush(prev, accs)

                bvp = jnp.full((L,), changed)
                nxt = [jnp.where(bvp, rw, a + rw)
                       for a, rw in zip(accs, [xbuf[rbase + r, pl.ds(cc * L, L)]
                                               for cc in range(CB)])]
                nxt.append(jnp.where(bvp, jnp.float32(1), accs[CB] + 1))
                sm[0] = br
                return tuple(nxt)

            return lax.fori_loop(0, L, row, tuple(accs))

        accs = lax.cond(b0 == b15, fastf, slowf, tuple(accs))
        sm[0] = b15
        return accs

    ngroups = GBASE + jnp.where(w < GEXTRA, 1, 0)
    init = tuple(jnp.zeros((L,), jnp.float32) for _ in range(CB + 1))
    accs = lax.fori_loop(0, ngroups, step, init)

    cur0 = sm[0]
    do_flush(jnp.where(cur0 < 0, DUMP - CNTB, cur0), list(accs))
    pltpu.sync_copy(flushbuf, sacc.at[idbuf], add=True)

    plsc.subcore_barrier()

    pltpu.sync_copy(sacc.at[pl.ds(sid * 16, 16)],
                    out_hbm.at[cid, pl.ds(sid * 16, 16)])


def _head_body(p_ref, u_ref, w_ref, b_ref, o_ref):
    s = p_ref[0] + p_ref[1]
    counts = s[CNTB:, 0:1]
    mean = s[:B] / jnp.maximum(counts, 1.0)
    w = w_ref[...]
    h = lax.dot_general(u_ref[...], w[:, :D], (((1,), (1,)), ((), ())),
                        preferred_element_type=jnp.float32)
    h = h + lax.dot_general(mean, w[:, D:], (((1,), (1,)), ((), ())),
                            preferred_element_type=jnp.float32)
    h = h + b_ref[...]
    o_ref[...] = jnp.maximum(h, 0.0)


_head = pl.pallas_call(
    _head_body,
    out_shape=jax.ShapeDtypeStruct((B, D), jnp.float32),
)


def kernel(x, edge_index, edge_attr, u, batch, W, b):
    del edge_index, edge_attr
    parts = _seg_sums(x, batch.astype(jnp.int32))
    return _head(parts, u, W, b.reshape(1, D))


# speedup vs baseline: 1.0312x; 1.0312x over previous
import functools

import jax
import jax.numpy as jnp
from jax import lax
from jax.experimental import pallas as pl
from jax.experimental.pallas import tpu as pltpu
from jax.experimental.pallas import tpu_sc as plsc

N = 10000
D = 128
B = 128
NC = 2
NS = 16
NW = NC * NS
L = 16
CB = D // L
NSC = 6144
GROUPS = NSC // L
GBASE = GROUPS // NW
RBASE = GBASE * L
CNTB = B
DUMP = 2 * B
SROWS = 264
CHUNK = 4
NTC = N - NSC
TCR = 31
TCPAD = TCR * 128 - NTC


def _iota():
    return lax.iota(jnp.int32, L)


_mesh = plsc.VectorSubcoreMesh(core_axis_name="c", subcore_axis_name="s")


@functools.partial(
    pl.kernel,
    compiler_params=pltpu.CompilerParams(needs_layout_passes=False),
    out_type=jax.ShapeDtypeStruct((NC, 2 * B, D), jnp.float32),
    mesh=_mesh,
    scratch_types=[
        pltpu.VMEM((RBASE, D), jnp.float32),
        pltpu.VMEM((RBASE + L,), jnp.int32),
        pltpu.VMEM((L, D), jnp.float32),
        pltpu.VMEM((L,), jnp.int32),
        pltpu.VMEM((8, D), jnp.float32),
        pltpu.SMEM((4,), jnp.int32),
        pltpu.SemaphoreType.DMA,
        pltpu.SemaphoreType.DMA,
        pltpu.VMEM_SHARED((SROWS, D), jnp.float32),
    ],
)
def _seg_sums(x_hbm, batch_hbm, out_hbm,
              xbuf, bbuf, flushbuf, idbuf, zbuf, sm, semx, semb, sacc):
    cid = lax.axis_index("c")
    sid = lax.axis_index("s")
    w = cid * NS + sid

    zero = jnp.zeros((L,), jnp.float32)
    dump_idv = jnp.full((L,), DUMP, jnp.int32)

    base = w * RBASE
    nchunks = RBASE // (CHUNK * L)
    for k in range(nchunks):
        pltpu.async_copy(x_hbm.at[pl.ds(base + k * CHUNK * L, CHUNK * L)],
                         xbuf.at[pl.ds(k * CHUNK * L, CHUNK * L)], semx)
    bcp = pltpu.async_copy(batch_hbm.at[pl.ds(base, RBASE)],
                           bbuf.at[pl.ds(0, RBASE)], semb)

    for r in range(8):
        for cc in range(CB):
            zbuf[r, pl.ds(cc * L, L)] = zero
    sm[0] = jnp.int32(-1)
    sm[1] = jnp.int32(0)
    idbuf[...] = dump_idv

    pltpu.sync_copy(zbuf, sacc.at[pl.ds(sid * 16, 8)])
    pltpu.sync_copy(zbuf, sacc.at[pl.ds(sid * 16 + 8, 8)])

    @pl.when(sid == 0)
    def _():
        pltpu.sync_copy(zbuf, sacc.at[pl.ds(256, 8)])

    plsc.subcore_barrier()
    bcp.wait()

    def do_flush(seg, accs):
        row_s = sm[1] % L
        rv0 = jnp.full((L,), row_s, jnp.int32)
        rv1 = rv0 + 1
        for cc in range(CB):
            plsc.store_scatter(flushbuf, [rv0, cc * L + _iota()], accs[cc])
            plsc.store_scatter(flushbuf, [rv1, cc * L + _iota()], accs[CB])
        idv = idbuf[...]
        idv = jnp.where(_iota() == row_s, jnp.full((L,), seg, jnp.int32), idv)
        idv = jnp.where(_iota() == row_s + 1,
                        jnp.full((L,), seg + CNTB, jnp.int32), idv)
        idbuf[...] = idv

        @pl.when(row_s == L - 2)
        def _():
            pltpu.sync_copy(flushbuf, sacc.at[idbuf], add=True)
            idbuf[...] = dump_idv

        sm[1] = sm[1] + 2

    def step(g, accs):
        accs = list(accs)
        rbase = g * L

        @pl.when(g % CHUNK == 0)
        def _():
            pltpu.make_async_copy(x_hbm.at[pl.ds(0, CHUNK * L)],
                                  xbuf.at[pl.ds(0, CHUNK * L)], semx).wait()

        bv = bbuf[pl.ds(rbase, L)]
        b0 = bv[0]
        b15 = bv[L - 1]
        cur0 = sm[0]

        def fastf(accs):
            @pl.when(cur0 != b0)
            def _():
                @pl.when(cur0 >= 0)
                def _():
                    do_flush(cur0, accs)

            bvp = jnp.full((L,), cur0 != b0)
            run = [jnp.where(bvp, 0.0, a) for a in accs]
            for r in range(L):
                for cc in range(CB):
                    run[cc] = run[cc] + xbuf[rbase + r, pl.ds(cc * L, L)]
            run[CB] = run[CB] + jnp.float32(L)
            return tuple(run)

        def slowf(accs):
            def row(r, accs):
                accs = list(accs)
                br = bbuf[pl.ds(rbase + r, L)][0]
                prev = sm[0]
                changed = prev != br

                @pl.when(changed)
                def _():
                    @pl.when(prev >= 0)
                    def _():
                        do_flush(prev, accs)

                bvp = jnp.full((L,), changed)
                nxt = [jnp.where(bvp, rw, a + rw)
                       for a, rw in zip(accs, [xbuf[rbase + r, pl.ds(cc * L, L)]
                                               for cc in range(CB)])]
                nxt.append(jnp.where(bvp, jnp.float32(1), accs[CB] + 1))
                sm[0] = br
                return tuple(nxt)

            return lax.fori_loop(0, L, row, tuple(accs))

        accs = lax.cond(b0 == b15, fastf, slowf, tuple(accs))
        sm[0] = b15
        return accs

    init = tuple(jnp.zeros((L,), jnp.float32) for _ in range(CB + 1))
    accs = lax.fori_loop(0, GBASE, step, init)

    cur0 = sm[0]
    do_flush(jnp.where(cur0 < 0, DUMP - CNTB, cur0), list(accs))
    pltpu.sync_copy(flushbuf, sacc.at[idbuf], add=True)

    plsc.subcore_barrier()

    pltpu.sync_copy(sacc.at[pl.ds(sid * 16, 16)],
                    out_hbm.at[cid, pl.ds(sid * 16, 16)])


def _tc_partial_body(x_ref, b_ref, sums_ref, cnt_ref):
    ids = b_ref[...]
    ri = (lax.broadcasted_iota(jnp.int32, (TCR, 128), 0) * 128
          + lax.broadcasted_iota(jnp.int32, (TCR, 128), 1))
    seg = lax.broadcasted_iota(jnp.int32, (TCR, 128, B), 2)
    onehot = jnp.where((ids[:, :, None] == seg) & (ri[:, :, None] < NTC),
                       jnp.float32(1), jnp.float32(0))
    onehot2 = onehot.reshape(TCR * 128, B)
    sums_ref[...] = lax.dot_general(
        onehot2, x_ref[...], (((0,), (0,)), ((), ())),
        preferred_element_type=jnp.float32)
    ones = jnp.ones((TCR * 128, 8), jnp.float32)
    cnt_ref[...] = lax.dot_general(
        onehot2, ones, (((0,), (0,)), ((), ())),
        preferred_element_type=jnp.float32)


_tc_partial = pl.pallas_call(
    _tc_partial_body,
    out_shape=(jax.ShapeDtypeStruct((B, D), jnp.float32),
               jax.ShapeDtypeStruct((B, 8), jnp.float32)),
)


def _head_body(p_ref, ts_ref, tc_ref, u_ref, w_ref, b_ref, o_ref):
    s = p_ref[0] + p_ref[1]
    counts = s[CNTB:, 0:1] + tc_ref[:, 0:1]
    mean = (s[:B] + ts_ref[...]) / jnp.maximum(counts, 1.0)
    w = w_ref[...]
    h = lax.dot_general(u_ref[...], w[:, :D], (((1,), (1,)), ((), ())),
                        preferred_element_type=jnp.float32)
    h = h + lax.dot_general(mean, w[:, D:], (((1,), (1,)), ((), ())),
                            preferred_element_type=jnp.float32)
    h = h + b_ref[...]
    o_ref[...] = jnp.maximum(h, 0.0)


_head = pl.pallas_call(
    _head_body,
    out_shape=jax.ShapeDtypeStruct((B, D), jnp.float32),
)


def kernel(x, edge_index, edge_attr, u, batch, W, b):
    del edge_index, edge_attr
    batch = batch.astype(jnp.int32)
    parts = _seg_sums(x, batch)
    x_tail = jnp.pad(x[NSC:], ((0, TCPAD), (0, 0)))
    b_tail = jnp.pad(batch[NSC:], (0, TCPAD)).reshape(TCR, 128)
    tc_sums, tc_cnt = _tc_partial(x_tail, b_tail)
    return _head(parts, tc_sums, tc_cnt, u, W, b.reshape(1, D))


# speedup vs baseline: 1.0530x; 1.0211x over previous
import functools

import jax
import jax.numpy as jnp
from jax import lax
from jax.experimental import pallas as pl
from jax.experimental.pallas import tpu as pltpu
from jax.experimental.pallas import tpu_sc as plsc

N = 10000
D = 128
B = 128
NC = 2
NS = 16
NW = NC * NS
L = 16
CB = D // L
NSC = 6144
GROUPS = NSC // L
GBASE = GROUPS // NW
RBASE = GBASE * L
CNTB = B
DUMP = 2 * B
SROWS = 264
CHUNK = 4
NTC = N - NSC
TCR = 31
TCPAD = TCR * 128 - NTC


def _iota():
    return lax.iota(jnp.int32, L)


_mesh = plsc.VectorSubcoreMesh(core_axis_name="c", subcore_axis_name="s")


@functools.partial(
    pl.kernel,
    compiler_params=pltpu.CompilerParams(needs_layout_passes=False),
    out_type=jax.ShapeDtypeStruct((NC, 2 * B, D), jnp.float32),
    mesh=_mesh,
    scratch_types=[
        pltpu.VMEM((RBASE * D,), jnp.float32),
        pltpu.VMEM((RBASE + L,), jnp.int32),
        pltpu.VMEM((L, D), jnp.float32),
        pltpu.VMEM((L,), jnp.int32),
        pltpu.VMEM((8, D), jnp.float32),
        pltpu.SMEM((4,), jnp.int32),
        pltpu.SemaphoreType.DMA,
        pltpu.SemaphoreType.DMA,
        pltpu.VMEM_SHARED((SROWS, D), jnp.float32),
    ],
)
def _seg_sums(x_hbm, batch_hbm, out_hbm,
              xbuf, bbuf, flushbuf, idbuf, zbuf, sm, semx, semb, sacc):
    cid = lax.axis_index("c")
    sid = lax.axis_index("s")
    w = cid * NS + sid

    zero = jnp.zeros((L,), jnp.float32)
    dump_idv = jnp.full((L,), DUMP, jnp.int32)

    base = w * RBASE
    nchunks = RBASE // (CHUNK * L)
    CW = CHUNK * L * D
    for k in range(nchunks):
        pltpu.async_copy(x_hbm.at[pl.ds(base * D + k * CW, CW)],
                         xbuf.at[pl.ds(k * CW, CW)], semx)
    bcp = pltpu.async_copy(batch_hbm.at[pl.ds(base, RBASE)],
                           bbuf.at[pl.ds(0, RBASE)], semb)

    for r in range(8):
        for cc in range(CB):
            zbuf[r, pl.ds(cc * L, L)] = zero
    sm[0] = jnp.int32(-1)
    sm[1] = jnp.int32(0)
    idbuf[...] = dump_idv

    pltpu.sync_copy(zbuf, sacc.at[pl.ds(sid * 16, 8)])
    pltpu.sync_copy(zbuf, sacc.at[pl.ds(sid * 16 + 8, 8)])

    @pl.when(sid == 0)
    def _():
        pltpu.sync_copy(zbuf, sacc.at[pl.ds(256, 8)])

    plsc.subcore_barrier()
    bcp.wait()

    def do_flush(seg, accs):
        row_s = sm[1] % L
        rv0 = jnp.full((L,), row_s, jnp.int32)
        rv1 = rv0 + 1
        for cc in range(CB):
            plsc.store_scatter(flushbuf, [rv0, cc * L + _iota()], accs[cc])
            plsc.store_scatter(flushbuf, [rv1, cc * L + _iota()], accs[CB])
        idv = idbuf[...]
        idv = jnp.where(_iota() == row_s, jnp.full((L,), seg, jnp.int32), idv)
        idv = jnp.where(_iota() == row_s + 1,
                        jnp.full((L,), seg + CNTB, jnp.int32), idv)
        idbuf[...] = idv

        @pl.when(row_s == L - 2)
        def _():
            pltpu.sync_copy(flushbuf, sacc.at[idbuf], add=True)
            idbuf[...] = dump_idv

        sm[1] = sm[1] + 2

    def step(g, accs):
        accs = list(accs)
        rbase = g * L

        @pl.when(g % CHUNK == 0)
        def _():
            pltpu.make_async_copy(x_hbm.at[pl.ds(0, CHUNK * L * D)],
                                  xbuf.at[pl.ds(0, CHUNK * L * D)], semx).wait()

        bv = bbuf[pl.ds(rbase, L)]
        b0 = bv[0]
        b15 = bv[L - 1]
        cur0 = sm[0]

        def fastf(accs):
            @pl.when(cur0 != b0)
            def _():
                @pl.when(cur0 >= 0)
                def _():
                    do_flush(cur0, accs)

            bvp = jnp.full((L,), cur0 != b0)
            run = [jnp.where(bvp, 0.0, a) for a in accs]
            goff = rbase * D
            for r in range(L):
                for cc in range(CB):
                    run[cc] = run[cc] + xbuf[pl.ds(goff + r * D + cc * L, L)]
            run[CB] = run[CB] + jnp.float32(L)
            return tuple(run)

        def slowf(accs):
            def row(r, accs):
                accs = list(accs)
                br = bbuf[pl.ds(rbase + r, L)][0]
                prev = sm[0]
                changed = prev != br

                @pl.when(changed)
                def _():
                    @pl.when(prev >= 0)
                    def _():
                        do_flush(prev, accs)

                bvp = jnp.full((L,), changed)
                roff = (rbase + r) * D
                nxt = [jnp.where(bvp, rw, a + rw)
                       for a, rw in zip(accs, [xbuf[pl.ds(roff + cc * L, L)]
                                               for cc in range(CB)])]
                nxt.append(jnp.where(bvp, jnp.float32(1), accs[CB] + 1))
                sm[0] = br
                return tuple(nxt)

            return lax.fori_loop(0, L, row, tuple(accs))

        accs = lax.cond(b0 == b15, fastf, slowf, tuple(accs))
        sm[0] = b15
        return accs

    init = tuple(jnp.zeros((L,), jnp.float32) for _ in range(CB + 1))
    accs = lax.fori_loop(0, GBASE, step, init)

    cur0 = sm[0]
    do_flush(jnp.where(cur0 < 0, DUMP - CNTB, cur0), list(accs))
    pltpu.sync_copy(flushbuf, sacc.at[idbuf], add=True)

    plsc.subcore_barrier()

    pltpu.sync_copy(sacc.at[pl.ds(sid * 16, 16)],
                    out_hbm.at[cid, pl.ds(sid * 16, 16)])


def _tc_partial_body(x_ref, b_ref, sums_ref, cnt_ref):
    ids = b_ref[...]
    seg = lax.broadcasted_iota(jnp.int32, (NTC // L, L, B), 2)
    onehot = jnp.where(ids[:, :, None] == seg, jnp.float32(1), jnp.float32(0))
    onehot2 = onehot.reshape(NTC, B)
    sums_ref[...] = lax.dot_general(
        onehot2, x_ref[...], (((0,), (0,)), ((), ())),
        preferred_element_type=jnp.float32)
    ones = jnp.ones((NTC, 8), jnp.float32)
    cnt_ref[...] = lax.dot_general(
        onehot2, ones, (((0,), (0,)), ((), ())),
        preferred_element_type=jnp.float32)


_tc_partial = pl.pallas_call(
    _tc_partial_body,
    out_shape=(jax.ShapeDtypeStruct((B, D), jnp.float32),
               jax.ShapeDtypeStruct((B, 8), jnp.float32)),
)


def _head_body(p_ref, ts_ref, tc_ref, u_ref, w_ref, b_ref, o_ref):
    s = p_ref[0] + p_ref[1]
    counts = s[CNTB:, 0:1] + tc_ref[:, 0:1]
    mean = (s[:B] + ts_ref[...]) / jnp.maximum(counts, 1.0)
    w = w_ref[...]
    h = lax.dot_general(u_ref[...], w[:, :D], (((1,), (1,)), ((), ())),
                        preferred_element_type=jnp.float32)
    h = h + lax.dot_general(mean, w[:, D:], (((1,), (1,)), ((), ())),
                            preferred_element_type=jnp.float32)
    h = h + b_ref[...]
    o_ref[...] = jnp.maximum(h, 0.0)


_head = pl.pallas_call(
    _head_body,
    out_shape=jax.ShapeDtypeStruct((B, D), jnp.float32),
)


def kernel(x, edge_index, edge_attr, u, batch, W, b):
    del edge_index, edge_attr
    batch = batch.astype(jnp.int32)
    parts = _seg_sums(x.reshape(-1), batch)
    x_tail = x[NSC:]
    b_tail = batch[NSC:].reshape(NTC // L, L)
    tc_sums, tc_cnt = _tc_partial(x_tail, b_tail)
    return _head(parts, tc_sums, tc_cnt, u, W, b.reshape(1, D))


# speedup vs baseline: 1.0538x; 1.0007x over previous
import functools

import jax
import jax.numpy as jnp
from jax import lax
from jax.experimental import pallas as pl
from jax.experimental.pallas import tpu as pltpu
from jax.experimental.pallas import tpu_sc as plsc

N = 10000
D = 128
B = 128
NC = 2
NS = 16
NW = NC * NS
L = 16
CB = D // L
NSC = 6144
GROUPS = NSC // L
GBASE = GROUPS // NW
RBASE = GBASE * L
CNTB = B
DUMP = 2 * B
SROWS = 264
CHUNK = 2
NTC = N - NSC
TCR = 31
TCPAD = TCR * 128 - NTC


def _iota():
    return lax.iota(jnp.int32, L)


_mesh = plsc.VectorSubcoreMesh(core_axis_name="c", subcore_axis_name="s")


@functools.partial(
    pl.kernel,
    compiler_params=pltpu.CompilerParams(needs_layout_passes=False),
    out_type=jax.ShapeDtypeStruct((NC, 2 * B, D), jnp.float32),
    mesh=_mesh,
    scratch_types=[
        pltpu.VMEM((RBASE * D,), jnp.float32),
        pltpu.VMEM((RBASE + L,), jnp.int32),
        pltpu.VMEM((L, D), jnp.float32),
        pltpu.VMEM((L,), jnp.int32),
        pltpu.VMEM((8, D), jnp.float32),
        pltpu.SMEM((4,), jnp.int32),
        pltpu.SemaphoreType.DMA,
        pltpu.SemaphoreType.DMA,
        pltpu.VMEM_SHARED((SROWS, D), jnp.float32),
    ],
)
def _seg_sums(x_hbm, batch_hbm, out_hbm,
              xbuf, bbuf, flushbuf, idbuf, zbuf, sm, semx, semb, sacc):
    cid = lax.axis_index("c")
    sid = lax.axis_index("s")
    w = cid * NS + sid

    zero = jnp.zeros((L,), jnp.float32)
    dump_idv = jnp.full((L,), DUMP, jnp.int32)

    base = w * RBASE
    nchunks = RBASE // (CHUNK * L)
    CW = CHUNK * L * D
    for k in range(nchunks):
        pltpu.async_copy(x_hbm.at[pl.ds(base * D + k * CW, CW)],
                         xbuf.at[pl.ds(k * CW, CW)], semx)
    bcp = pltpu.async_copy(batch_hbm.at[pl.ds(base, RBASE)],
                           bbuf.at[pl.ds(0, RBASE)], semb)

    for r in range(8):
        for cc in range(CB):
            zbuf[r, pl.ds(cc * L, L)] = zero
    sm[0] = jnp.int32(-1)
    sm[1] = jnp.int32(0)
    idbuf[...] = dump_idv

    pltpu.sync_copy(zbuf, sacc.at[pl.ds(sid * 16, 8)])
    pltpu.sync_copy(zbuf, sacc.at[pl.ds(sid * 16 + 8, 8)])

    @pl.when(sid == 0)
    def _():
        pltpu.sync_copy(zbuf, sacc.at[pl.ds(256, 8)])

    plsc.subcore_barrier()
    bcp.wait()

    def do_flush(seg, accs):
        row_s = sm[1] % L
        rv0 = jnp.full((L,), row_s, jnp.int32)
        rv1 = rv0 + 1
        for cc in range(CB):
            plsc.store_scatter(flushbuf, [rv0, cc * L + _iota()], accs[cc])
            plsc.store_scatter(flushbuf, [rv1, cc * L + _iota()], accs[CB])
        idv = idbuf[...]
        idv = jnp.where(_iota() == row_s, jnp.full((L,), seg, jnp.int32), idv)
        idv = jnp.where(_iota() == row_s + 1,
                        jnp.full((L,), seg + CNTB, jnp.int32), idv)
        idbuf[...] = idv

        @pl.when(row_s == L - 2)
        def _():
            pltpu.sync_copy(flushbuf, sacc.at[idbuf], add=True)
            idbuf[...] = dump_idv

        sm[1] = sm[1] + 2

    def step(g, accs):
        accs = list(accs)
        rbase = g * L

        @pl.when(g % CHUNK == 0)
        def _():
            pltpu.make_async_copy(x_hbm.at[pl.ds(0, CHUNK * L * D)],
                                  xbuf.at[pl.ds(0, CHUNK * L * D)], semx).wait()

        bv = bbuf[pl.ds(rbase, L)]
        b0 = bv[0]
        b15 = bv[L - 1]
        cur0 = sm[0]

        def fastf(accs):
            @pl.when(cur0 != b0)
            def _():
                @pl.when(cur0 >= 0)
                def _():
                    do_flush(cur0, accs)

            bvp = jnp.full((L,), cur0 != b0)
            run = [jnp.where(bvp, 0.0, a) for a in accs]
            goff = rbase * D
            for r in range(L):
                for cc in range(CB):
                    run[cc] = run[cc] + xbuf[pl.ds(goff + r * D + cc * L, L)]
            run[CB] = run[CB] + jnp.float32(L)
            return tuple(run)

        def slowf(accs):
            def row(r, accs):
                accs = list(accs)
                br = bbuf[pl.ds(rbase + r, L)][0]
                prev = sm[0]
                changed = prev != br

                @pl.when(changed)
                def _():
                    @pl.when(prev >= 0)
                    def _():
                        do_flush(prev, accs)

                bvp = jnp.full((L,), changed)
                roff = (rbase + r) * D
                nxt = [jnp.where(bvp, rw, a + rw)
                       for a, rw in zip(accs, [xbuf[pl.ds(roff + cc * L, L)]
                                               for cc in range(CB)])]
                nxt.append(jnp.where(bvp, jnp.float32(1), accs[CB] + 1))
                sm[0] = br
                return tuple(nxt)

            return lax.fori_loop(0, L, row, tuple(accs))

        accs = lax.cond(b0 == b15, fastf, slowf, tuple(accs))
        sm[0] = b15
        return accs

    init = tuple(jnp.zeros((L,), jnp.float32) for _ in range(CB + 1))
    accs = lax.fori_loop(0, GBASE, step, init)

    cur0 = sm[0]
    do_flush(jnp.where(cur0 < 0, DUMP - CNTB, cur0), list(accs))
    pltpu.sync_copy(flushbuf, sacc.at[idbuf], add=True)

    plsc.subcore_barrier()

    pltpu.sync_copy(sacc.at[pl.ds(sid * 16, 16)],
                    out_hbm.at[cid, pl.ds(sid * 16, 16)])


def _tc_partial_body(x_ref, b_ref, sums_ref, cnt_ref):
    ids = b_ref[...]
    seg = lax.broadcasted_iota(jnp.int32, (NTC // L, L, B), 2)
    onehot = jnp.where(ids[:, :, None] == seg, jnp.float32(1), jnp.float32(0))
    onehot2 = onehot.reshape(NTC, B)
    sums_ref[...] = lax.dot_general(
        onehot2, x_ref[...], (((0,), (0,)), ((), ())),
        preferred_element_type=jnp.float32)
    ones = jnp.ones((NTC, 8), jnp.float32)
    cnt_ref[...] = lax.dot_general(
        onehot2, ones, (((0,), (0,)), ((), ())),
        preferred_element_type=jnp.float32)


_tc_partial = pl.pallas_call(
    _tc_partial_body,
    out_shape=(jax.ShapeDtypeStruct((B, D), jnp.float32),
               jax.ShapeDtypeStruct((B, 8), jnp.float32)),
)


def _head_body(p_ref, ts_ref, tc_ref, u_ref, w_ref, b_ref, o_ref):
    s = p_ref[0] + p_ref[1]
    counts = s[CNTB:, 0:1] + tc_ref[:, 0:1]
    mean = (s[:B] + ts_ref[...]) / jnp.maximum(counts, 1.0)
    w = w_ref[...]
    h = lax.dot_general(u_ref[...], w[:, :D], (((1,), (1,)), ((), ())),
                        preferred_element_type=jnp.float32)
    h = h + lax.dot_general(mean, w[:, D:], (((1,), (1,)), ((), ())),
                            preferred_element_type=jnp.float32)
    h = h + b_ref[...]
    o_ref[...] = jnp.maximum(h, 0.0)


_head = pl.pallas_call(
    _head_body,
    out_shape=jax.ShapeDtypeStruct((B, D), jnp.float32),
)


def kernel(x, edge_index, edge_attr, u, batch, W, b):
    del edge_index, edge_attr
    batch = batch.astype(jnp.int32)
    parts = _seg_sums(x.reshape(-1), batch)
    x_tail = x[NSC:]
    b_tail = batch[NSC:].reshape(NTC // L, L)
    tc_sums, tc_cnt = _tc_partial(x_tail, b_tail)
    return _head(parts, tc_sums, tc_cnt, u, W, b.reshape(1, D))
